# Initial kernel scaffold; baseline (speedup 1.0000x reference)
import jax
import jax.numpy as jnp
from jax.experimental import pallas as pl


def kernel(q, k, v):
    raise NotImplementedError("write your pallas kernel here")


# speedup vs baseline: 11.9646x; 11.9646x over previous
import functools

import jax
import jax.numpy as jnp
from jax import lax
from jax.experimental import pallas as pl
from jax.experimental.pallas import tpu as pltpu
from jax.experimental.pallas import tpu_sc as plsc

B, S, D = 4, 2048, 768
W = 5
NWIN = S // W
L = NWIN * W
LANES = 16
NWORKERS = 32
NTASKS = B * (D // LANES)
TASKS_PER_WORKER = NTASKS // NWORKERS
GROUPS = D // LANES

_NET = ((0, 1), (3, 4), (2, 4), (2, 3), (0, 3), (0, 2), (1, 4), (1, 3), (1, 2))


def _body(v_hbm, out_hbm, in_v, out_v):
    cid = lax.axis_index("c")
    sid = lax.axis_index("s")
    wid = sid * 2 + cid
    lane = lax.broadcasted_iota(jnp.int32, (LANES,), 0)

    for task in range(TASKS_PER_WORKER):
        tid = wid * TASKS_PER_WORKER + task
        b = tid // GROUPS
        c0 = (tid - b * GROUPS) * LANES
        i_vec = c0 + lane

        pltpu.sync_copy(v_hbm.at[b, :, pl.ds(c0, LANES)], in_v)

        def window(k, carry, i_vec=i_vec):
            base = i_vec + W * k
            xs = []
            for j in range(W):
                row = (base + j) & (S - 1)
                xs.append(plsc.load_gather(in_v, [row, lane]))
            for a, bb in _NET:
                lo = jnp.minimum(xs[a], xs[bb])
                hi = jnp.maximum(xs[a], xs[bb])
                xs[a], xs[bb] = lo, hi
            t0 = base
            for r in range(W):
                t = t0 + r
                t = jnp.where(t >= L, t - L, t)
                plsc.store_scatter(out_v, [t, lane], xs[r])
            return carry

        lax.fori_loop(0, NWIN, window, 0)

        pltpu.sync_copy(out_v, out_hbm.at[b, :, pl.ds(c0, LANES)])


@jax.jit
def _run(v):
    kfn = functools.partial(
        pl.kernel,
        mesh=plsc.VectorSubcoreMesh(core_axis_name="c", subcore_axis_name="s"),
        out_type=jax.ShapeDtypeStruct((B, L, D), jnp.float32),
        scratch_types=[
            pltpu.VMEM((S, LANES), jnp.float32),
            pltpu.VMEM((L, LANES), jnp.float32),
        ],
        compiler_params=pltpu.CompilerParams(
            use_tc_tiling_on_sc=False, needs_layout_passes=False
        ),
    )(_body)
    return kfn(v)


def kernel(q, k, v):
    del q, k
    return _run(v)


# speedup vs baseline: 12.2935x; 1.0275x over previous
import functools

import jax
import jax.numpy as jnp
from jax import lax
from jax.experimental import pallas as pl
from jax.experimental.pallas import tpu as pltpu
from jax.experimental.pallas import tpu_sc as plsc

B, S, D = 4, 2048, 768
W = 5
NWIN = S // W
L = NWIN * W
LANES = 16
NWORKERS = 32
NTASKS = B * (D // LANES)
TASKS_PER_WORKER = NTASKS // NWORKERS
GROUPS = D // LANES

_NET = ((0, 1), (3, 4), (2, 4), (2, 3), (0, 3), (0, 2), (1, 4), (1, 3), (1, 2))


def _body(v_hbm, out_hbm, in_v, out_v):
    cid = lax.axis_index("c")
    sid = lax.axis_index("s")
    wid = sid * 2 + cid
    lane = lax.broadcasted_iota(jnp.int32, (LANES,), 0)
    zero16 = jnp.zeros((LANES,), jnp.float32)

    for task in range(TASKS_PER_WORKER):
        tid = wid * TASKS_PER_WORKER + task
        b = tid // GROUPS
        c0 = (tid - b * GROUPS) * LANES
        i_vec = c0 + lane

        pltpu.sync_copy(v_hbm.at[b, :, pl.ds(c0, LANES)], in_v.at[pl.ds(0, S)])
        pltpu.sync_copy(v_hbm.at[b, pl.ds(0, 4), pl.ds(c0, LANES)],
                        in_v.at[pl.ds(S, 4)])

        for e in range(4):
            out_v[e, :] = zero16
            out_v[L + e, :] = zero16

        @plsc.parallel_loop(0, NWIN, unroll=8)
        def window(k, i_vec=i_vec):
            base = i_vec + W * k
            m = base & (S - 1)
            t0 = jnp.where(base >= L, base - L, base)
            xs = [plsc.load_gather(in_v, [m + j, lane]) for j in range(W)]
            for a, bb in _NET:
                lo = jnp.minimum(xs[a], xs[bb])
                hi = jnp.maximum(xs[a], xs[bb])
                xs[a], xs[bb] = lo, hi
            for r in range(W):
                plsc.store_scatter(out_v, [t0 + r, lane], xs[r])

        for e in range(4):
            out_v[e, :] = out_v[e, :] + out_v[L + e, :]

        pltpu.sync_copy(out_v.at[pl.ds(0, L)], out_hbm.at[b, :, pl.ds(c0, LANES)])


@jax.jit
def _run(v):
    kfn = functools.partial(
        pl.kernel,
        mesh=plsc.VectorSubcoreMesh(core_axis_name="c", subcore_axis_name="s"),
        out_type=jax.ShapeDtypeStruct((B, L, D), jnp.float32),
        scratch_types=[
            pltpu.VMEM((S + 4, LANES), jnp.float32),
            pltpu.VMEM((L + 4, LANES), jnp.float32),
        ],
        compiler_params=pltpu.CompilerParams(
            use_tc_tiling_on_sc=False, needs_layout_passes=False
        ),
    )(_body)
    return kfn(v)


def kernel(q, k, v):
    del q, k
    return _run(v)
